# baseline (speedup 1.0000x reference)
import functools

import jax
import jax.numpy as jnp
from jax import lax
from jax.experimental import pallas as pl
from jax.experimental.pallas import tpu as pltpu
from jax.experimental.pallas import tpu_sc as plsc

NUM_ARMS = 1000000
D = 32
B = 4096
S = 50
N = B * S

NC = 2
NS = 16
NW = NC * NS
TOK_PER_W = N // NW
CHUNK = 128
NCHUNK = TOK_PER_W // CHUNK


def _sc_gather_body(action_hbm, table_hbm, out_hbm, idx_v, buf0, buf1,
                    sem0, sem1):
  wid = lax.axis_index("s") * NC + lax.axis_index("c")
  tokbase = wid * TOK_PER_W
  pltpu.sync_copy(action_hbm.at[pl.ds(tokbase, TOK_PER_W)], idx_v)

  bufs = (buf0, buf1)
  sems = (sem0, sem1)

  def chunk_idx(j):
    return idx_v.at[pl.ds(j * CHUNK, CHUNK)]

  pltpu.async_copy(table_hbm.at[chunk_idx(0)], buf0, sem0)
  pltpu.async_copy(table_hbm.at[chunk_idx(1)], buf1, sem1)

  def step(i, _):
    base = i * 2
    for b in range(2):
      j = base + b
      pltpu.make_async_copy(table_hbm.at[chunk_idx(j)], bufs[b], sems[b]).wait()
      pltpu.sync_copy(bufs[b], out_hbm.at[pl.ds(tokbase + j * CHUNK, CHUNK)])
      @pl.when(j + 2 < NCHUNK)
      def _():
        pltpu.async_copy(table_hbm.at[chunk_idx(j + 2)], bufs[b], sems[b])
    return 0

  lax.fori_loop(0, NCHUNK // 2, step, 0)


_sc_gather = functools.partial(
    pl.kernel,
    out_type=jax.ShapeDtypeStruct((N, D), jnp.float32),
    mesh=plsc.VectorSubcoreMesh(core_axis_name="c", subcore_axis_name="s"),
    scratch_types=[
        pltpu.VMEM((TOK_PER_W,), jnp.int32),
        pltpu.VMEM((CHUNK, D), jnp.float32),
        pltpu.VMEM((CHUNK, D), jnp.float32),
        pltpu.SemaphoreType.DMA,
        pltpu.SemaphoreType.DMA,
    ],
    compiler_params=pltpu.CompilerParams(use_tc_tiling_on_sc=False),
)(_sc_gather_body)


def _tc_assemble_body(state_ref, reward_ref, act_ref, wo_ref, bo_ref,
                      wr_ref, br_ref, out_ref):
  obs = state_ref[...] * wo_ref[...] + bo_ref[...]
  rew = reward_ref[...] * wr_ref[...] + br_ref[...]
  out_ref[...] = jnp.concatenate([obs, act_ref[...], rew], axis=-1)


TBLK = 2048


def _tc_assemble(state_flat, reward_flat, act_c, W_obs, b_obs, W_rew, b_rew):
  grid = (N // TBLK,)
  return pl.pallas_call(
      _tc_assemble_body,
      grid=grid,
      in_specs=[
          pl.BlockSpec((TBLK, 1), lambda i: (i, 0)),
          pl.BlockSpec((TBLK, 1), lambda i: (i, 0)),
          pl.BlockSpec((TBLK, D), lambda i: (i, 0)),
          pl.BlockSpec((1, D), lambda i: (0, 0)),
          pl.BlockSpec((1, D), lambda i: (0, 0)),
          pl.BlockSpec((1, D), lambda i: (0, 0)),
          pl.BlockSpec((1, D), lambda i: (0, 0)),
      ],
      out_specs=pl.BlockSpec((TBLK, 3 * D), lambda i: (i, 0)),
      out_shape=jax.ShapeDtypeStruct((N, 3 * D), jnp.float32),
  )(state_flat, reward_flat, act_c, W_obs, b_obs, W_rew, b_rew)


@jax.jit
def kernel(state, action, reward, W_obs, b_obs, emb_table, W_rew, b_rew):
  action_v = action.astype(jnp.int32).reshape(N)
  act_c = _sc_gather(action_v, emb_table)
  out = _tc_assemble(
      state.reshape(N, 1),
      reward.reshape(N, 1),
      act_c,
      W_obs,
      b_obs.reshape(1, D),
      W_rew,
      b_rew.reshape(1, D),
  )
  return out.reshape(B, 3 * S, D)


# speedup vs baseline: 1.5483x; 1.5483x over previous
import functools

import jax
import jax.numpy as jnp
from jax import lax
from jax.experimental import pallas as pl
from jax.experimental.pallas import tpu as pltpu
from jax.experimental.pallas import tpu_sc as plsc

NUM_ARMS = 1000000
D = 32
B = 4096
S = 50
N = B * S

NC = 2
NS = 16
NW = NC * NS
CHUNK = B // NW


def _sc_gather_body(action_hbm, table_hbm, out_hbm, idx_v, buf0, buf1,
                    sem0, sem1):
  wid = lax.axis_index("s") * NC + lax.axis_index("c")
  bbase = wid * CHUNK
  pltpu.sync_copy(action_hbm.at[:, pl.ds(bbase, CHUNK)], idx_v)

  bufs = (buf0, buf1)
  sems = (sem0, sem1)

  pltpu.async_copy(table_hbm.at[idx_v.at[0]], buf0, sem0)
  pltpu.async_copy(table_hbm.at[idx_v.at[1]], buf1, sem1)

  def step(i, _):
    base = i * 2
    for b in range(2):
      s = base + b
      pltpu.make_async_copy(table_hbm.at[idx_v.at[s]], bufs[b], sems[b]).wait()
      pltpu.sync_copy(bufs[b], out_hbm.at[pl.ds(s * B + bbase, CHUNK)])
      @pl.when(s + 2 < S)
      def _():
        pltpu.async_copy(table_hbm.at[idx_v.at[s + 2]], bufs[b], sems[b])
    return 0

  lax.fori_loop(0, S // 2, step, 0)


_sc_gather = functools.partial(
    pl.kernel,
    out_type=jax.ShapeDtypeStruct((N, D), jnp.float32),
    mesh=plsc.VectorSubcoreMesh(core_axis_name="c", subcore_axis_name="s"),
    scratch_types=[
        pltpu.VMEM((S, CHUNK), jnp.int32),
        pltpu.VMEM((CHUNK, D), jnp.float32),
        pltpu.VMEM((CHUNK, D), jnp.float32),
        pltpu.SemaphoreType.DMA,
        pltpu.SemaphoreType.DMA,
    ],
    compiler_params=pltpu.CompilerParams(use_tc_tiling_on_sc=False),
)(_sc_gather_body)


def _tc_assemble_body(state_ref, reward_ref, act_ref, wo_ref, bo_ref,
                      wr_ref, br_ref, out_ref):
  wo = jnp.transpose(wo_ref[...])
  bo = jnp.transpose(bo_ref[...])
  wr = jnp.transpose(wr_ref[...])
  br = jnp.transpose(br_ref[...])
  st = state_ref[0]
  rw = reward_ref[0]
  out_ref[0] = wo * st + bo
  out_ref[1] = jnp.transpose(act_ref[0])
  out_ref[2] = wr * rw + br


TBB = 1024


def _tc_assemble(state_t, reward_t, act_c, W_obs, b_obs, W_rew, b_rew):
  grid = (S, B // TBB)
  return pl.pallas_call(
      _tc_assemble_body,
      grid=grid,
      in_specs=[
          pl.BlockSpec((1, 1, TBB), lambda s, j: (s, 0, j)),
          pl.BlockSpec((1, 1, TBB), lambda s, j: (s, 0, j)),
          pl.BlockSpec((1, TBB, D), lambda s, j: (s, j, 0)),
          pl.BlockSpec((1, D), lambda s, j: (0, 0)),
          pl.BlockSpec((1, D), lambda s, j: (0, 0)),
          pl.BlockSpec((1, D), lambda s, j: (0, 0)),
          pl.BlockSpec((1, D), lambda s, j: (0, 0)),
      ],
      out_specs=pl.BlockSpec((3, D, TBB), lambda s, j: (s, 0, j)),
      out_shape=jax.ShapeDtypeStruct((3 * S, D, B), jnp.float32),
  )(state_t, reward_t, act_c, W_obs, b_obs, W_rew, b_rew)


@jax.jit
def kernel(state, action, reward, W_obs, b_obs, emb_table, W_rew, b_rew):
  action_t = action.astype(jnp.int32).T
  state_t = state.transpose(1, 2, 0)
  reward_t = reward.T.reshape(S, 1, B)
  act_c = _sc_gather(action_t, emb_table)
  out_t = _tc_assemble(
      state_t,
      reward_t,
      act_c.reshape(S, B, D),
      W_obs,
      b_obs.reshape(1, D),
      W_rew,
      b_rew.reshape(1, D),
  )
  return out_t.transpose(2, 0, 1)
